# Initial kernel scaffold; baseline (speedup 1.0000x reference)
import jax
import jax.numpy as jnp
from jax.experimental import pallas as pl


def kernel(x, edge_index, edge_attr, Wn, bn, Wl1, bl1, Wr1, Wl2, bl2, Wr2, We, be, W1, b1, W2, b2, Ws, bs):
    raise NotImplementedError("write your pallas kernel here")


# speedup vs baseline: 4.0839x; 4.0839x over previous
import functools

import jax
import jax.numpy as jnp
from jax import lax
from jax.experimental import pallas as pl
from jax.experimental.pallas import tpu as pltpu
from jax.experimental.pallas import tpu_sc as plsc

_F32 = jnp.float32
_NC = 2
_NS = 16
_NW = _NC * _NS
_CH = 128
_SC_PARAMS = pltpu.CompilerParams(use_tc_tiling_on_sc=False)


def _worker_id():
    return lax.axis_index("s") * _NC + lax.axis_index("c")


def _sc_mesh():
    return plsc.VectorSubcoreMesh(core_axis_name="c", subcore_axis_name="s",
                                  num_cores=_NC, num_subcores=_NS)


def _sc_segsum(g, src3d, dst3d, zeros64, zeros16, ones128, with_cnt):
    NP, H = g.shape
    nch = src3d.shape[0]
    nsr = NP // _NS

    out_type = [jax.ShapeDtypeStruct((2 * NP, H), _F32)]
    scratch = [
        pltpu.VMEM((1, _CH), jnp.int32),
        pltpu.VMEM((1, _CH), jnp.int32),
        pltpu.VMEM((_CH, H), _F32),
        pltpu.VMEM_SHARED((NP, H), _F32),
    ]
    if with_cnt:
        out_type.append(jax.ShapeDtypeStruct((2 * NP, 16), _F32))
        scratch += [
            pltpu.VMEM((_CH, 16), _F32),
            pltpu.VMEM_SHARED((NP, 16), _F32),
        ]

    @functools.partial(pl.kernel, out_type=out_type, mesh=_sc_mesh(),
                       scratch_types=scratch, compiler_params=_SC_PARAMS)
    def k(*refs):
        if with_cnt:
            (g_hbm, src_hbm, dst_hbm, z64_hbm, z16_hbm, ones_hbm,
             p_hbm, cnt_hbm, srcv, dstv, rows, acc, onesv, cacc) = refs
        else:
            (g_hbm, src_hbm, dst_hbm, z64_hbm,
             p_hbm, srcv, dstv, rows, acc) = refs
        c = lax.axis_index("c")
        s = lax.axis_index("s")
        wid = s * _NC + c
        pltpu.sync_copy(z64_hbm.at[pl.ds(s * nsr, nsr)],
                        acc.at[pl.ds(s * nsr, nsr)])
        if with_cnt:
            pltpu.sync_copy(z16_hbm.at[pl.ds(s * nsr, nsr)],
                            cacc.at[pl.ds(s * nsr, nsr)])
            pltpu.sync_copy(ones_hbm, onesv)
        plsc.subcore_barrier()

        @pl.loop(wid, nch, step=_NW)
        def _(chunk):
            pltpu.sync_copy(src_hbm.at[chunk], srcv)
            pltpu.sync_copy(dst_hbm.at[chunk], dstv)
            pltpu.sync_copy(g_hbm.at[srcv.at[0]], rows)
            pltpu.sync_copy(rows, acc.at[dstv.at[0]], add=True)
            if with_cnt:
                pltpu.sync_copy(onesv, cacc.at[dstv.at[0]], add=True)

        plsc.subcore_barrier()
        pltpu.sync_copy(acc.at[pl.ds(s * nsr, nsr)],
                        p_hbm.at[pl.ds(c * NP + s * nsr, nsr)])
        if with_cnt:
            pltpu.sync_copy(cacc.at[pl.ds(s * nsr, nsr)],
                            cnt_hbm.at[pl.ds(c * NP + s * nsr, nsr)])

    if with_cnt:
        return k(g, src3d, dst3d, zeros64, zeros16, ones128)
    return k(g, src3d, dst3d, zeros64)


def _sc_edge_gather_add(A, B, src3d, dst3d):
    NP, H = A.shape
    nch = src3d.shape[0]
    E = nch * _CH

    @functools.partial(
        pl.kernel,
        out_type=jax.ShapeDtypeStruct((E, H), _F32),
        mesh=_sc_mesh(),
        compiler_params=_SC_PARAMS,
        scratch_types=[
            pltpu.VMEM((1, _CH), jnp.int32),
            pltpu.VMEM((1, _CH), jnp.int32),
            pltpu.VMEM((_CH, H), _F32),
            pltpu.VMEM((_CH, H), _F32),
        ],
    )
    def k(a_hbm, b_hbm, src_hbm, dst_hbm, t_hbm, srcv, dstv, arows, brows):
        wid = _worker_id()

        @pl.loop(wid, nch, step=_NW)
        def _(chunk):
            pltpu.sync_copy(src_hbm.at[chunk], srcv)
            pltpu.sync_copy(dst_hbm.at[chunk], dstv)
            pltpu.sync_copy(a_hbm.at[srcv.at[0]], arows)
            pltpu.sync_copy(b_hbm.at[dstv.at[0]], brows)

            @pl.loop(0, _CH)
            def _(i):
                for j in range(0, H, 16):
                    arows[i, pl.ds(j, 16)] = (arows[i, pl.ds(j, 16)]
                                              + brows[i, pl.ds(j, 16)])

            pltpu.sync_copy(arows, t_hbm.at[pl.ds(chunk * _CH, _CH)])

    return k(A, B, src3d, dst3d)


def _tc_stage1(x, Wn, bn, Wl1, Wr1):
    NP, D = x.shape
    H = Wn.shape[1]
    BN = NP // 8

    def body(x_ref, wn_ref, bn_ref, wl_ref, wr_ref, g_ref, r_ref):
        h0 = jnp.maximum(x_ref[...] @ wn_ref[...] + bn_ref[...], 0.0)
        g_ref[...] = h0 @ wl_ref[...]
        r_ref[...] = h0 @ wr_ref[...]

    return pl.pallas_call(
        body,
        grid=(NP // BN,),
        in_specs=[
            pl.BlockSpec((BN, D), lambda i: (i, 0)),
            pl.BlockSpec((D, H), lambda i: (0, 0)),
            pl.BlockSpec((1, H), lambda i: (0, 0)),
            pl.BlockSpec((H, H), lambda i: (0, 0)),
            pl.BlockSpec((H, H), lambda i: (0, 0)),
        ],
        out_specs=[pl.BlockSpec((BN, H), lambda i: (i, 0))] * 2,
        out_shape=[jax.ShapeDtypeStruct((NP, H), _F32)] * 2,
    )(x, Wn, bn, Wl1, Wr1)


def _tc_conv_combine(P, CNT, r, bl, Wl, Wr):
    twoNP, H = P.shape
    NP = twoNP // 2
    BN = NP // 8
    NB = NP // BN

    def body(p0, p1, c0r, c1r, r_ref, bl_ref, wl_ref, wr_ref, g_ref, r2_ref):
        S = p0[...] + p1[...]
        cnt = c0r[...][:, 0:1] + c1r[...][:, 0:1]
        h = jnp.maximum(S / jnp.maximum(cnt, 1.0) + bl_ref[...] + r_ref[...],
                        0.0)
        g_ref[...] = h @ wl_ref[...]
        r2_ref[...] = h @ wr_ref[...]

    return pl.pallas_call(
        body,
        grid=(NB,),
        in_specs=[
            pl.BlockSpec((BN, H), lambda i: (i, 0)),
            pl.BlockSpec((BN, H), lambda i: (i + NB, 0)),
            pl.BlockSpec((BN, 16), lambda i: (i, 0)),
            pl.BlockSpec((BN, 16), lambda i: (i + NB, 0)),
            pl.BlockSpec((BN, H), lambda i: (i, 0)),
            pl.BlockSpec((1, H), lambda i: (0, 0)),
            pl.BlockSpec((H, H), lambda i: (0, 0)),
            pl.BlockSpec((H, H), lambda i: (0, 0)),
        ],
        out_specs=[pl.BlockSpec((BN, H), lambda i: (i, 0))] * 2,
        out_shape=[jax.ShapeDtypeStruct((NP, H), _F32)] * 2,
    )(P, P, CNT, CNT, r, bl, Wl, Wr)


def _tc_score(T, ea, Wf, cvec, vv, c0):
    E, H = T.shape
    De = ea.shape[1]
    BE = 3200
    NB = E // BE

    def body(t_ref, ea_ref, wf_ref, cv_ref, v_ref, c0_ref, o_ref):
        t = t_ref[...] + ea_ref[...] @ wf_ref[...] + cv_ref[...]
        o_ref[...] = jnp.maximum(t, 0.0) @ v_ref[...] + c0_ref[...]

    return pl.pallas_call(
        body,
        grid=(NB,),
        in_specs=[
            pl.BlockSpec((BE, H), lambda i: (i, 0)),
            pl.BlockSpec((BE, De), lambda i: (i, 0)),
            pl.BlockSpec((De, H), lambda i: (0, 0)),
            pl.BlockSpec((1, H), lambda i: (0, 0)),
            pl.BlockSpec((H, 1), lambda i: (0, 0)),
            pl.BlockSpec((1, 1), lambda i: (0, 0)),
        ],
        out_specs=pl.BlockSpec((BE, 1), lambda i: (i, 0)),
        out_shape=jax.ShapeDtypeStruct((E, 1), _F32),
    )(T, ea, Wf, cvec, vv, c0)


def kernel(x, edge_index, edge_attr, Wn, bn, Wl1, bl1, Wr1, Wl2, bl2, Wr2,
           We, be, W1, b1, W2, b2, Ws, bs):
    N = x.shape[0]
    H = Wn.shape[1]
    NP = ((N + 127) // 128) * 128

    src3d = edge_index[0].reshape(-1, 1, _CH)
    dst3d = edge_index[1].reshape(-1, 1, _CH)
    x_p = jnp.pad(x, ((0, NP - N), (0, 0)))
    zeros64 = jnp.zeros((NP, H), _F32)
    zeros16 = jnp.zeros((NP, 16), _F32)
    ones128 = jnp.ones((_CH, 16), _F32)

    W1a, W1b, W1c = W1[:H], W1[H:2 * H], W1[2 * H:]
    Wf = We @ W1c
    cvec = (be @ W1c + b1).reshape(1, H)
    vv = W2 @ Ws
    c0 = (b2 @ Ws + bs).reshape(1, 1)

    g1, r1 = _tc_stage1(x_p, Wn, bn.reshape(1, H), Wl1, Wr1)
    P1, CNT = _sc_segsum(g1, src3d, dst3d, zeros64, zeros16, ones128,
                         with_cnt=True)
    g2, r2 = _tc_conv_combine(P1, CNT, r1, bl1.reshape(1, H), Wl2, Wr2)
    (P2,) = _sc_segsum(g2, src3d, dst3d, zeros64, zeros16, ones128,
                       with_cnt=False)
    A, B = _tc_conv_combine(P2, CNT, r2, bl2.reshape(1, H), W1a, W1b)
    T = _sc_edge_gather_add(A, B, src3d, dst3d)
    return _tc_score(T, edge_attr, Wf, cvec, vv, c0)


# speedup vs baseline: 5.8033x; 1.4210x over previous
import functools

import jax
import jax.numpy as jnp
from jax import lax
from jax.experimental import pallas as pl
from jax.experimental.pallas import tpu as pltpu
from jax.experimental.pallas import tpu_sc as plsc

_F32 = jnp.float32
_NC = 2
_NS = 16
_NW = _NC * _NS
_CH = 128
_SC_PARAMS = pltpu.CompilerParams(use_tc_tiling_on_sc=False)


def _sc_mesh():
    return plsc.VectorSubcoreMesh(core_axis_name="c", subcore_axis_name="s",
                                  num_cores=_NC, num_subcores=_NS)


def _sc_segsum(g, src_g, dst_g, zeros64, zeros16, ones128, with_cnt):
    NP, H = g.shape
    ngr, G, _ = src_g.shape
    nsr = NP // _NS

    out_type = [jax.ShapeDtypeStruct((2 * NP, H), _F32)]
    scratch = [
        pltpu.VMEM((G, _CH), jnp.int32),
        pltpu.VMEM((G, _CH), jnp.int32),
        pltpu.VMEM((G * _CH, H), _F32),
        pltpu.VMEM_SHARED((NP, H), _F32),
        pltpu.SemaphoreType.DMA,
    ]
    if with_cnt:
        out_type.append(jax.ShapeDtypeStruct((2 * NP, 16), _F32))
        scratch += [
            pltpu.VMEM((_CH, 16), _F32),
            pltpu.VMEM_SHARED((NP, 16), _F32),
        ]

    @functools.partial(pl.kernel, out_type=out_type, mesh=_sc_mesh(),
                       scratch_types=scratch, compiler_params=_SC_PARAMS)
    def k(*refs):
        if with_cnt:
            (g_hbm, src_hbm, dst_hbm, z64_hbm, z16_hbm, ones_hbm,
             p_hbm, cnt_hbm, srcv, dstv, rows, acc, sem, onesv, cacc) = refs
        else:
            (g_hbm, src_hbm, dst_hbm, z64_hbm,
             p_hbm, srcv, dstv, rows, acc, sem) = refs
        c = lax.axis_index("c")
        s = lax.axis_index("s")
        wid = s * _NC + c
        pltpu.sync_copy(z64_hbm.at[pl.ds(s * nsr, nsr)],
                        acc.at[pl.ds(s * nsr, nsr)])
        if with_cnt:
            pltpu.sync_copy(z16_hbm.at[pl.ds(s * nsr, nsr)],
                            cacc.at[pl.ds(s * nsr, nsr)])
            pltpu.sync_copy(ones_hbm, onesv)
        plsc.subcore_barrier()

        @pl.loop(wid, ngr, step=_NW)
        def _(grp):
            pltpu.sync_copy(src_hbm.at[grp], srcv)
            pltpu.sync_copy(dst_hbm.at[grp], dstv)
            gathers = [
                pltpu.async_copy(g_hbm.at[srcv.at[j]],
                                 rows.at[pl.ds(j * _CH, _CH)], sem)
                for j in range(G)
            ]
            for h in gathers:
                h.wait()
            adds = [
                pltpu.async_copy(rows.at[pl.ds(j * _CH, _CH)],
                                 acc.at[dstv.at[j]], sem, add=True)
                for j in range(G)
            ]
            if with_cnt:
                adds += [
                    pltpu.async_copy(onesv, cacc.at[dstv.at[j]], sem,
                                     add=True)
                    for j in range(G)
                ]
            for h in adds:
                h.wait()

        plsc.subcore_barrier()
        pltpu.sync_copy(acc.at[pl.ds(s * nsr, nsr)],
                        p_hbm.at[pl.ds(c * NP + s * nsr, nsr)])
        if with_cnt:
            pltpu.sync_copy(cacc.at[pl.ds(s * nsr, nsr)],
                            cnt_hbm.at[pl.ds(c * NP + s * nsr, nsr)])

    if with_cnt:
        return k(g, src_g, dst_g, zeros64, zeros16, ones128)
    return k(g, src_g, dst_g, zeros64)


def _sc_edge_gather_add(A, B, src_g, dst_g, ident):
    NP, H = A.shape
    ngr, G, _ = src_g.shape
    E = ngr * G * _CH
    slab = G * _CH

    @functools.partial(
        pl.kernel,
        out_type=jax.ShapeDtypeStruct((E, H), _F32),
        mesh=_sc_mesh(),
        compiler_params=_SC_PARAMS,
        scratch_types=[
            pltpu.VMEM((G, _CH), jnp.int32),
            pltpu.VMEM((G, _CH), jnp.int32),
            pltpu.VMEM((G, _CH), jnp.int32),
            pltpu.VMEM((slab, H), _F32),
            pltpu.VMEM((slab, H), _F32),
            pltpu.VMEM_SHARED((_NS * slab, H), _F32),
            pltpu.SemaphoreType.DMA,
        ],
    )
    def k(a_hbm, b_hbm, src_hbm, dst_hbm, id_hbm, t_hbm,
          srcv, dstv, identv, arows, brows, tslab, sem):
        c = lax.axis_index("c")
        s = lax.axis_index("s")
        wid = s * _NC + c
        pltpu.sync_copy(id_hbm.at[s], identv)

        @pl.loop(wid, ngr, step=_NW)
        def _(grp):
            pltpu.sync_copy(src_hbm.at[grp], srcv)
            pltpu.sync_copy(dst_hbm.at[grp], dstv)
            gathers = [
                pltpu.async_copy(a_hbm.at[srcv.at[j]],
                                 arows.at[pl.ds(j * _CH, _CH)], sem)
                for j in range(G)
            ] + [
                pltpu.async_copy(b_hbm.at[dstv.at[j]],
                                 brows.at[pl.ds(j * _CH, _CH)], sem)
                for j in range(G)
            ]
            for h in gathers:
                h.wait()
            pltpu.sync_copy(arows, tslab.at[pl.ds(s * slab, slab)])
            adds = [
                pltpu.async_copy(brows.at[pl.ds(j * _CH, _CH)],
                                 tslab.at[identv.at[j]], sem, add=True)
                for j in range(G)
            ]
            for h in adds:
                h.wait()
            pltpu.sync_copy(tslab.at[pl.ds(s * slab, slab)],
                            t_hbm.at[pl.ds(grp * G * _CH, G * _CH)])

    return k(A, B, src_g, dst_g, ident)


def _tc_stage1(x, Wn, bn, Wl1, Wr1, NP):
    N, D = x.shape
    H = Wn.shape[1]
    BN = 2000

    def body(x_ref, wn_ref, bn_ref, wl_ref, wr_ref, g_ref, r_ref):
        h0 = jnp.maximum(x_ref[...] @ wn_ref[...] + bn_ref[...], 0.0)
        g_ref[...] = h0 @ wl_ref[...]
        r_ref[...] = h0 @ wr_ref[...]

    return pl.pallas_call(
        body,
        grid=(N // BN,),
        in_specs=[
            pl.BlockSpec((BN, D), lambda i: (i, 0)),
            pl.BlockSpec((D, H), lambda i: (0, 0)),
            pl.BlockSpec((1, H), lambda i: (0, 0)),
            pl.BlockSpec((H, H), lambda i: (0, 0)),
            pl.BlockSpec((H, H), lambda i: (0, 0)),
        ],
        out_specs=[pl.BlockSpec((BN, H), lambda i: (i, 0))] * 2,
        out_shape=[jax.ShapeDtypeStruct((NP, H), _F32)] * 2,
    )(x, Wn, bn, Wl1, Wr1)


def _tc_conv_combine(P, CNT, r, bl, Wl, Wr):
    twoNP, H = P.shape
    NP = twoNP // 2
    BN = NP // 8
    NB = NP // BN

    def body(p0, p1, c0r, c1r, r_ref, bl_ref, wl_ref, wr_ref, g_ref, r2_ref):
        S = p0[...] + p1[...]
        cnt = c0r[...][:, 0:1] + c1r[...][:, 0:1]
        h = jnp.maximum(S / jnp.maximum(cnt, 1.0) + bl_ref[...] + r_ref[...],
                        0.0)
        g_ref[...] = h @ wl_ref[...]
        r2_ref[...] = h @ wr_ref[...]

    return pl.pallas_call(
        body,
        grid=(NB,),
        in_specs=[
            pl.BlockSpec((BN, H), lambda i: (i, 0)),
            pl.BlockSpec((BN, H), lambda i: (i + NB, 0)),
            pl.BlockSpec((BN, 16), lambda i: (i, 0)),
            pl.BlockSpec((BN, 16), lambda i: (i + NB, 0)),
            pl.BlockSpec((BN, H), lambda i: (i, 0)),
            pl.BlockSpec((1, H), lambda i: (0, 0)),
            pl.BlockSpec((H, H), lambda i: (0, 0)),
            pl.BlockSpec((H, H), lambda i: (0, 0)),
        ],
        out_specs=[pl.BlockSpec((BN, H), lambda i: (i, 0))] * 2,
        out_shape=[jax.ShapeDtypeStruct((NP, H), _F32)] * 2,
    )(P, P, CNT, CNT, r, bl, Wl, Wr)


def _tc_score(T, ea, Wf, cvec, vv, c0):
    E, H = T.shape
    De = ea.shape[1]
    BE = 3200
    NB = E // BE

    def body(t_ref, ea_ref, wf_ref, cv_ref, v_ref, c0_ref, o_ref):
        t = t_ref[...] + ea_ref[...] @ wf_ref[...] + cv_ref[...]
        o_ref[...] = jnp.maximum(t, 0.0) @ v_ref[...] + c0_ref[...]

    return pl.pallas_call(
        body,
        grid=(NB,),
        in_specs=[
            pl.BlockSpec((BE, H), lambda i: (i, 0)),
            pl.BlockSpec((BE, De), lambda i: (i, 0)),
            pl.BlockSpec((De, H), lambda i: (0, 0)),
            pl.BlockSpec((1, H), lambda i: (0, 0)),
            pl.BlockSpec((H, 1), lambda i: (0, 0)),
            pl.BlockSpec((1, 1), lambda i: (0, 0)),
        ],
        out_specs=pl.BlockSpec((BE, 1), lambda i: (i, 0)),
        out_shape=jax.ShapeDtypeStruct((E, 1), _F32),
    )(T, ea, Wf, cvec, vv, c0)


def kernel(x, edge_index, edge_attr, Wn, bn, Wl1, bl1, Wr1, Wl2, bl2, Wr2,
           We, be, W1, b1, W2, b2, Ws, bs):
    N = x.shape[0]
    H = Wn.shape[1]
    NP = ((N + 127) // 128) * 128

    G1, G2, GE = 5, 10, 5
    src = edge_index[0]
    dst = edge_index[1]
    src_g1 = src.reshape(-1, G1, _CH)
    dst_g1 = dst.reshape(-1, G1, _CH)
    src_g2 = src.reshape(-1, G2, _CH)
    dst_g2 = dst.reshape(-1, G2, _CH)
    src_ge = src.reshape(-1, GE, _CH)
    dst_ge = dst.reshape(-1, GE, _CH)
    ident = (jnp.arange(_CH, dtype=jnp.int32)[None, None, :]
             + _CH * jnp.arange(GE, dtype=jnp.int32)[None, :, None]
             + GE * _CH * jnp.arange(_NS, dtype=jnp.int32)[:, None, None])
    zeros64 = jnp.zeros((NP, H), _F32)
    zeros16 = jnp.zeros((NP, 16), _F32)
    ones128 = jnp.ones((_CH, 16), _F32)

    W1a, W1b, W1c = W1[:H], W1[H:2 * H], W1[2 * H:]
    Wf = We @ W1c
    cvec = (be @ W1c + b1).reshape(1, H)
    vv = W2 @ Ws
    c0 = (b2 @ Ws + bs).reshape(1, 1)

    g1, r1 = _tc_stage1(x, Wn, bn.reshape(1, H), Wl1, Wr1, NP)
    P1, CNT = _sc_segsum(g1, src_g1, dst_g1, zeros64, zeros16, ones128,
                         with_cnt=True)
    g2, r2 = _tc_conv_combine(P1, CNT, r1, bl1.reshape(1, H), Wl2, Wr2)
    (P2,) = _sc_segsum(g2, src_g2, dst_g2, zeros64, zeros16, ones128,
                       with_cnt=False)
    A, B = _tc_conv_combine(P2, CNT, r2, bl2.reshape(1, H), W1a, W1b)
    T = _sc_edge_gather_add(A, B, src_ge, dst_ge, ident)
    return _tc_score(T, edge_attr, Wf, cvec, vv, c0)
